# baseline (speedup 1.0000x reference)
import functools

import jax
import jax.numpy as jnp
from jax import lax
from jax.experimental import pallas as pl
from jax.experimental.pallas import tpu as pltpu
from jax.experimental.pallas import tpu_sc as plsc

_K = 8
_PAD = 16


def _net_merge(a, b):
    if len(a) == 1:
        lo = jnp.minimum(a[0], b[0])
        hi = jnp.maximum(a[0], b[0])
        return [lo, hi]
    even = _net_merge(a[0::2], b[0::2])
    odd = _net_merge(a[1::2], b[1::2])
    res = [even[0]]
    for i in range(1, len(even)):
        res.append(jnp.minimum(odd[i - 1], even[i]))
        res.append(jnp.maximum(odd[i - 1], even[i]))
    res.append(odd[-1])
    return res


def _net_sort(xs):
    if len(xs) == 1:
        return xs
    h = len(xs) // 2
    return _net_merge(_net_sort(xs[:h]), _net_sort(xs[h:]))


def _knn_body(x_blk_ref, x_all_ref, idx_ref, d_ref):
    n = x_all_ref.shape[0]
    xb = x_blk_ref[...]
    xa = x_all_ref[...]
    a2a = jnp.sum(xa * xa, axis=1)[None, :]
    a2b = jnp.sum(xb * xb, axis=1, keepdims=True)
    g = lax.dot_general(xb, xa, (((1,), (1,)), ((), ())),
                        preferred_element_type=jnp.float32)
    d2 = jnp.maximum(a2b + a2a - 2.0 * g, 0.0)
    kb = lax.bitcast_convert_type(d2, jnp.int32)
    colsu = lax.broadcasted_iota(jnp.int32, d2.shape, 1)
    keys = jnp.bitwise_or(jnp.bitwise_and(kb, jnp.int32(-4096)), colsu)
    groups = [keys[:, g * 128:(g + 1) * 128] for g in range(n // 128)]
    cand = jnp.concatenate(_net_sort(groups)[:_K + 1], axis=1)
    idx_cols = []
    d_cols = []
    for _ in range(_K + 1):
        m = jnp.min(cand, axis=1, keepdims=True)
        idx_cols.append(jnp.bitwise_and(m, jnp.int32(4095)))
        d_cols.append(lax.bitcast_convert_type(
            jnp.bitwise_and(m, jnp.int32(-4096)), jnp.float32))
        cand = jnp.where(cand == m, jnp.int32(0x7FFFFFFF), cand)
    rb = xb.shape[0]
    pad = _PAD - (_K + 1)
    idx_ref[...] = jnp.concatenate(
        idx_cols + [jnp.full((rb, pad), -1, jnp.int32)], axis=1)
    d_ref[...] = jnp.concatenate(
        d_cols + [jnp.zeros((rb, pad), jnp.float32)], axis=1)


def _knn_topk(X, rb=2048):
    n, d_in = X.shape
    return pl.pallas_call(
        _knn_body,
        grid=(n // rb,),
        in_specs=[
            pl.BlockSpec((rb, d_in), lambda i: (i, 0)),
            pl.BlockSpec((n, d_in), lambda i: (0, 0)),
        ],
        out_specs=[
            pl.BlockSpec((rb, _PAD), lambda i: (i, 0)),
            pl.BlockSpec((rb, _PAD), lambda i: (i, 0)),
        ],
        out_shape=[
            jax.ShapeDtypeStruct((n, _PAD), jnp.int32),
            jax.ShapeDtypeStruct((n, _PAD), jnp.float32),
        ],
    )(X, X)


_SPAD = 16


def _edge_loss_call(nbr16, d16, Z):
    n, d_lat = Z.shape
    nc, ns, nl = 2, 16, 16
    nw = nc * ns
    rpw = n // nw
    epw = rpw * _K
    ch = 128
    nch = epw // ch
    gpc = ch // nl

    mesh = plsc.VectorSubcoreMesh(core_axis_name="c", subcore_axis_name="s")

    @functools.partial(
        pl.kernel,
        mesh=mesh,
        compiler_params=pltpu.CompilerParams(
            needs_layout_passes=False, use_tc_tiling_on_sc=False),
        out_type=jax.ShapeDtypeStruct((nw, nl), jnp.float32),
        scratch_types=[
            pltpu.VMEM((n * _SPAD,), jnp.int32),
            pltpu.VMEM((rpw * _SPAD,), jnp.float32),
            pltpu.VMEM((rpw, d_lat), jnp.float32),
            pltpu.VMEM((ch,), jnp.int32),
            pltpu.VMEM((ch, d_lat), jnp.float32),
            pltpu.VMEM((ch,), jnp.float32),
            pltpu.VMEM((nl,), jnp.float32),
            pltpu.SemaphoreType.DMA,
        ],
    )
    def launch(nbr_hbm, d_hbm, z_hbm, out_hbm,
               nbr_v, d_v, zi_v, jidx_v, zj_v, coef_v, sum_v, sem):
        cid = lax.axis_index("c")
        sid = lax.axis_index("s")
        wid = sid * nc + cid
        row0 = wid * rpw
        pltpu.sync_copy(nbr_hbm, nbr_v)
        pltpu.sync_copy(d_hbm.at[pl.ds(row0 * _SPAD, rpw * _SPAD)], d_v)
        pltpu.sync_copy(z_hbm.at[pl.ds(row0, rpw)], zi_v)

        lanes = lax.iota(jnp.int32, nl)

        def chunk_body(c, acc):
            ebase = c * ch
            for t in range(gpc):
                el = ebase + t * nl + lanes
                rl = lax.shift_right_logical(el, 3)
                cl = 1 + lax.bitwise_and(el, 7)
                rg = rl + row0
                jv = plsc.load_gather(nbr_v, [rg * _SPAD + cl])
                dv = plsc.load_gather(d_v, [rl * _SPAD + cl])
                w = jnp.exp(-dv)
                mut = jnp.zeros((nl,), jnp.int32)
                for cc in range(1, _K + 1):
                    cand = plsc.load_gather(nbr_v, [jv * _SPAD + cc])
                    mut = jnp.where(cand == rg, 1, mut)
                jidx_v[pl.ds(t * nl, nl)] = jv
                coef_v[pl.ds(t * nl, nl)] = w * (2.0 - mut.astype(jnp.float32))
            pltpu.async_copy(z_hbm.at[jidx_v], zj_v, sem).wait()

            def group_body(g, a):
                for u in range(nl):
                    e = g * nl + u
                    r = lax.shift_right_logical(ebase + e, 3)
                    cf = plsc.load_gather(
                        coef_v, [jnp.full((nl,), e, jnp.int32)])
                    for cc in range(d_lat // nl):
                        zi = zi_v[r, pl.ds(cc * nl, nl)]
                        zj_ = zj_v[e, pl.ds(cc * nl, nl)]
                        dlt = zi - zj_
                        a = a + cf * dlt * dlt
                return a

            return lax.fori_loop(0, gpc, group_body, acc)

        acc = lax.fori_loop(0, nch, chunk_body, jnp.zeros((nl,), jnp.float32))
        sum_v[...] = acc
        pltpu.sync_copy(sum_v, out_hbm.at[wid])

    return launch(nbr16.reshape(-1), d16.reshape(-1), Z)


def kernel(X, Z):
    n = X.shape[0]
    nbr16, d16 = _knn_topk(X)
    parts = _edge_loss_call(nbr16, d16, Z)
    return jnp.sum(parts) / (n * _K)


# speedup vs baseline: 1.2321x; 1.2321x over previous
import functools

import jax
import jax.numpy as jnp
from jax import lax
from jax.experimental import pallas as pl
from jax.experimental.pallas import tpu as pltpu
from jax.experimental.pallas import tpu_sc as plsc

_K = 8
_PAD = 16


def _net_merge(a, b):
    if len(a) == 1:
        lo = jnp.minimum(a[0], b[0])
        hi = jnp.maximum(a[0], b[0])
        return [lo, hi]
    even = _net_merge(a[0::2], b[0::2])
    odd = _net_merge(a[1::2], b[1::2])
    res = [even[0]]
    for i in range(1, len(even)):
        res.append(jnp.minimum(odd[i - 1], even[i]))
        res.append(jnp.maximum(odd[i - 1], even[i]))
    res.append(odd[-1])
    return res


def _net_sort(xs):
    if len(xs) == 1:
        return xs
    h = len(xs) // 2
    return _net_merge(_net_sort(xs[:h]), _net_sort(xs[h:]))


def _knn_body(x_blk_ref, x_all_ref, idx_ref, d_ref):
    n = x_all_ref.shape[0]
    xb = x_blk_ref[...]
    xa = x_all_ref[...]
    a2a = jnp.sum(xa * xa, axis=1)[None, :]
    a2b = jnp.sum(xb * xb, axis=1, keepdims=True)
    g = lax.dot_general(xb, xa, (((1,), (1,)), ((), ())),
                        preferred_element_type=jnp.float32)
    d2 = jnp.maximum(a2b + a2a - 2.0 * g, 0.0)
    kb = lax.bitcast_convert_type(d2, jnp.int32)
    colsu = lax.broadcasted_iota(jnp.int32, d2.shape, 1)
    keys = jnp.bitwise_or(jnp.bitwise_and(kb, jnp.int32(-4096)), colsu)
    groups = [keys[:, g * 128:(g + 1) * 128] for g in range(n // 128)]
    cand = jnp.concatenate(_net_sort(groups)[:_K + 1], axis=1)
    idx_cols = []
    d_cols = []
    for _ in range(_K + 1):
        m = jnp.min(cand, axis=1, keepdims=True)
        idx_cols.append(jnp.bitwise_and(m, jnp.int32(4095)))
        d_cols.append(lax.bitcast_convert_type(
            jnp.bitwise_and(m, jnp.int32(-4096)), jnp.float32))
        cand = jnp.where(cand == m, jnp.int32(0x7FFFFFFF), cand)
    rb = xb.shape[0]
    pad = _PAD - (_K + 1)
    idx_ref[...] = jnp.concatenate(
        idx_cols + [jnp.full((rb, pad), -1, jnp.int32)], axis=1)
    d_ref[...] = jnp.concatenate(
        d_cols + [jnp.zeros((rb, pad), jnp.float32)], axis=1)


def _knn_topk(X, rb=512):
    n, d_in = X.shape
    return pl.pallas_call(
        _knn_body,
        grid=(n // rb,),
        in_specs=[
            pl.BlockSpec((rb, d_in), lambda i: (i, 0)),
            pl.BlockSpec((n, d_in), lambda i: (0, 0)),
        ],
        out_specs=[
            pl.BlockSpec((rb, _PAD), lambda i: (i, 0)),
            pl.BlockSpec((rb, _PAD), lambda i: (i, 0)),
        ],
        out_shape=[
            jax.ShapeDtypeStruct((n, _PAD), jnp.int32),
            jax.ShapeDtypeStruct((n, _PAD), jnp.float32),
        ],
    )(X, X)


_SPAD = 16


def _edge_loss_call(nbr16, d16, Z):
    n, d_lat = Z.shape
    nc, ns, nl = 2, 16, 16
    nw = nc * ns
    rpw = n // nw
    epw = rpw * _K
    ch = 128
    nch = epw // ch
    gpc = ch // nl

    mesh = plsc.VectorSubcoreMesh(core_axis_name="c", subcore_axis_name="s")

    @functools.partial(
        pl.kernel,
        mesh=mesh,
        compiler_params=pltpu.CompilerParams(
            needs_layout_passes=False, use_tc_tiling_on_sc=False),
        out_type=jax.ShapeDtypeStruct((nw, nl), jnp.float32),
        scratch_types=[
            pltpu.VMEM((n * _SPAD,), jnp.int32),
            pltpu.VMEM((rpw * _SPAD,), jnp.float32),
            pltpu.VMEM((rpw, d_lat), jnp.float32),
            pltpu.VMEM((ch,), jnp.int32),
            pltpu.VMEM((ch, d_lat), jnp.float32),
            pltpu.VMEM((ch,), jnp.float32),
            pltpu.VMEM((nl,), jnp.float32),
            pltpu.SemaphoreType.DMA,
        ],
    )
    def launch(nbr_hbm, d_hbm, z_hbm, out_hbm,
               nbr_v, d_v, zi_v, jidx_v, zj_v, coef_v, sum_v, sem):
        cid = lax.axis_index("c")
        sid = lax.axis_index("s")
        wid = sid * nc + cid
        row0 = wid * rpw
        pltpu.sync_copy(nbr_hbm, nbr_v)
        pltpu.sync_copy(d_hbm.at[pl.ds(row0 * _SPAD, rpw * _SPAD)], d_v)
        pltpu.sync_copy(z_hbm.at[pl.ds(row0, rpw)], zi_v)

        lanes = lax.iota(jnp.int32, nl)

        def chunk_body(c, acc):
            ebase = c * ch
            for t in range(gpc):
                el = ebase + t * nl + lanes
                rl = lax.shift_right_logical(el, 3)
                cl = 1 + lax.bitwise_and(el, 7)
                rg = rl + row0
                jv = plsc.load_gather(nbr_v, [rg * _SPAD + cl])
                dv = plsc.load_gather(d_v, [rl * _SPAD + cl])
                w = jnp.exp(-dv)
                mut = jnp.zeros((nl,), jnp.int32)
                for cc in range(1, _K + 1):
                    cand = plsc.load_gather(nbr_v, [jv * _SPAD + cc])
                    mut = jnp.where(cand == rg, 1, mut)
                jidx_v[pl.ds(t * nl, nl)] = jv
                coef_v[pl.ds(t * nl, nl)] = w * (2.0 - mut.astype(jnp.float32))
            pltpu.async_copy(z_hbm.at[jidx_v], zj_v, sem).wait()

            def group_body(g, a):
                for u in range(nl):
                    e = g * nl + u
                    r = lax.shift_right_logical(ebase + e, 3)
                    cf = plsc.load_gather(
                        coef_v, [jnp.full((nl,), e, jnp.int32)])
                    for cc in range(d_lat // nl):
                        zi = zi_v[r, pl.ds(cc * nl, nl)]
                        zj_ = zj_v[e, pl.ds(cc * nl, nl)]
                        dlt = zi - zj_
                        a = a + cf * dlt * dlt
                return a

            return lax.fori_loop(0, gpc, group_body, acc)

        acc = lax.fori_loop(0, nch, chunk_body, jnp.zeros((nl,), jnp.float32))
        sum_v[...] = acc
        pltpu.sync_copy(sum_v, out_hbm.at[wid])

    return launch(nbr16.reshape(-1), d16.reshape(-1), Z)


def kernel(X, Z):
    n = X.shape[0]
    nbr16, d16 = _knn_topk(X)
    parts = _edge_loss_call(nbr16, d16, Z)
    return jnp.sum(parts) / (n * _K)


# speedup vs baseline: 1.2537x; 1.0175x over previous
import functools

import jax
import jax.numpy as jnp
from jax import lax
from jax.experimental import pallas as pl
from jax.experimental.pallas import tpu as pltpu
from jax.experimental.pallas import tpu_sc as plsc

_K = 8
_PAD = 16


def _net_merge(a, b):
    if len(a) == 1:
        lo = jnp.minimum(a[0], b[0])
        hi = jnp.maximum(a[0], b[0])
        return [lo, hi]
    even = _net_merge(a[0::2], b[0::2])
    odd = _net_merge(a[1::2], b[1::2])
    res = [even[0]]
    for i in range(1, len(even)):
        res.append(jnp.minimum(odd[i - 1], even[i]))
        res.append(jnp.maximum(odd[i - 1], even[i]))
    res.append(odd[-1])
    return res


def _net_sort(xs):
    if len(xs) == 1:
        return xs
    h = len(xs) // 2
    return _net_merge(_net_sort(xs[:h]), _net_sort(xs[h:]))


def _knn_body(x_blk_ref, x_all_ref, idx_ref, d_ref):
    n = x_all_ref.shape[0]
    xb = x_blk_ref[...]
    xa = x_all_ref[...]
    a2a = jnp.sum(xa * xa, axis=1)[None, :]
    a2b = jnp.sum(xb * xb, axis=1, keepdims=True)
    g = lax.dot_general(xb, xa, (((1,), (1,)), ((), ())),
                        preferred_element_type=jnp.float32)
    d2 = jnp.maximum(a2b + a2a - 2.0 * g, 0.0)
    kb = lax.bitcast_convert_type(d2, jnp.int32)
    colsu = lax.broadcasted_iota(jnp.int32, d2.shape, 1)
    keys = jnp.bitwise_or(jnp.bitwise_and(kb, jnp.int32(-4096)), colsu)
    groups = [keys[:, g * 128:(g + 1) * 128] for g in range(n // 128)]
    cand = jnp.concatenate(_net_sort(groups)[:_K + 1], axis=1)
    idx_cols = []
    d_cols = []
    for _ in range(_K + 1):
        m = jnp.min(cand, axis=1, keepdims=True)
        idx_cols.append(jnp.bitwise_and(m, jnp.int32(4095)))
        d_cols.append(lax.bitcast_convert_type(
            jnp.bitwise_and(m, jnp.int32(-4096)), jnp.float32))
        cand = jnp.where(cand == m, jnp.int32(0x7FFFFFFF), cand)
    rb = xb.shape[0]
    pad = _PAD - (_K + 1)
    idx_ref[...] = jnp.concatenate(
        idx_cols + [jnp.full((rb, pad), -1, jnp.int32)], axis=1)
    d_ref[...] = jnp.concatenate(
        d_cols + [jnp.zeros((rb, pad), jnp.float32)], axis=1)


def _knn_topk(X, rb=1024):
    n, d_in = X.shape
    return pl.pallas_call(
        _knn_body,
        grid=(n // rb,),
        in_specs=[
            pl.BlockSpec((rb, d_in), lambda i: (i, 0)),
            pl.BlockSpec((n, d_in), lambda i: (0, 0)),
        ],
        out_specs=[
            pl.BlockSpec((rb, _PAD), lambda i: (i, 0)),
            pl.BlockSpec((rb, _PAD), lambda i: (i, 0)),
        ],
        out_shape=[
            jax.ShapeDtypeStruct((n, _PAD), jnp.int32),
            jax.ShapeDtypeStruct((n, _PAD), jnp.float32),
        ],
    )(X, X)


_SPAD = 16


def _edge_loss_call(nbr16, d16, Z):
    n, d_lat = Z.shape
    nc, ns, nl = 2, 16, 16
    nw = nc * ns
    rpw = n // nw
    epw = rpw * _K
    ch = 128
    nch = epw // ch
    gpc = ch // nl

    mesh = plsc.VectorSubcoreMesh(core_axis_name="c", subcore_axis_name="s")

    @functools.partial(
        pl.kernel,
        mesh=mesh,
        compiler_params=pltpu.CompilerParams(
            needs_layout_passes=False, use_tc_tiling_on_sc=False),
        out_type=jax.ShapeDtypeStruct((nw, nl), jnp.float32),
        scratch_types=[
            pltpu.VMEM((n * _SPAD,), jnp.int32),
            pltpu.VMEM((rpw * _SPAD,), jnp.float32),
            pltpu.VMEM((rpw, d_lat), jnp.float32),
            pltpu.VMEM((ch,), jnp.int32),
            pltpu.VMEM((ch, d_lat), jnp.float32),
            pltpu.VMEM((ch,), jnp.float32),
            pltpu.VMEM((nl,), jnp.float32),
            pltpu.SemaphoreType.DMA,
        ],
    )
    def launch(nbr_hbm, d_hbm, z_hbm, out_hbm,
               nbr_v, d_v, zi_v, jidx_v, zj_v, coef_v, sum_v, sem):
        cid = lax.axis_index("c")
        sid = lax.axis_index("s")
        wid = sid * nc + cid
        row0 = wid * rpw
        pltpu.sync_copy(nbr_hbm, nbr_v)
        pltpu.sync_copy(d_hbm.at[pl.ds(row0 * _SPAD, rpw * _SPAD)], d_v)
        pltpu.sync_copy(z_hbm.at[pl.ds(row0, rpw)], zi_v)

        lanes = lax.iota(jnp.int32, nl)

        def chunk_body(c, acc):
            ebase = c * ch
            for t in range(gpc):
                el = ebase + t * nl + lanes
                rl = lax.shift_right_logical(el, 3)
                cl = 1 + lax.bitwise_and(el, 7)
                rg = rl + row0
                jv = plsc.load_gather(nbr_v, [rg * _SPAD + cl])
                dv = plsc.load_gather(d_v, [rl * _SPAD + cl])
                w = jnp.exp(-dv)
                mut = jnp.zeros((nl,), jnp.int32)
                for cc in range(1, _K + 1):
                    cand = plsc.load_gather(nbr_v, [jv * _SPAD + cc])
                    mut = jnp.where(cand == rg, 1, mut)
                jidx_v[pl.ds(t * nl, nl)] = jv
                coef_v[pl.ds(t * nl, nl)] = w * (2.0 - mut.astype(jnp.float32))
            pltpu.async_copy(z_hbm.at[jidx_v], zj_v, sem).wait()

            def group_body(g, a):
                for u in range(nl):
                    e = g * nl + u
                    r = lax.shift_right_logical(ebase + e, 3)
                    cf = plsc.load_gather(
                        coef_v, [jnp.full((nl,), e, jnp.int32)])
                    for cc in range(d_lat // nl):
                        zi = zi_v[r, pl.ds(cc * nl, nl)]
                        zj_ = zj_v[e, pl.ds(cc * nl, nl)]
                        dlt = zi - zj_
                        a = a + cf * dlt * dlt
                return a

            return lax.fori_loop(0, gpc, group_body, acc)

        acc = lax.fori_loop(0, nch, chunk_body, jnp.zeros((nl,), jnp.float32))
        sum_v[...] = acc
        pltpu.sync_copy(sum_v, out_hbm.at[wid])

    return launch(nbr16.reshape(-1), d16.reshape(-1), Z)


def kernel(X, Z):
    n = X.shape[0]
    nbr16, d16 = _knn_topk(X)
    parts = _edge_loss_call(nbr16, d16, Z)
    return jnp.sum(parts) / (n * _K)
